# Initial kernel scaffold; baseline (speedup 1.0000x reference)
import jax
import jax.numpy as jnp
from jax.experimental import pallas as pl


def kernel(preds, labels, mask, w_values_dict, adj_matrix):
    raise NotImplementedError("write your pallas kernel here")


# speedup vs baseline: 9.2810x; 9.2810x over previous
import jax
import jax.numpy as jnp
import numpy as np
from jax.experimental import pallas as pl

_N = 1024
_C = 8
_GAMMA = 1.0
_PER = 0.001
_SIG1 = float(1.0 / (1.0 + np.exp(-1.0)))


def _loss_body(preds_ref, lab_ref, maskf_ref, a_ref, diag_ref, out_ref):
    preds = preds_ref[...]
    labels = lab_ref[...]
    maskf = maskf_ref[...]
    a_bf = a_ref[...]
    diag_row = diag_ref[...]

    mx = jnp.max(preds, axis=1, keepdims=True)
    lse = jnp.log(jnp.sum(jnp.exp(preds - mx), axis=1, keepdims=True)) + mx
    logp = preds - lse
    cls_iota = jax.lax.broadcasted_iota(jnp.int32, (_N, _C), 1)
    lab_oh = (cls_iota == labels).astype(jnp.float32)
    ce = -jnp.sum(logp * lab_oh) * (1.0 / _N)

    m_cls = lab_oh * maskf
    ncnt = jnp.sum(m_cls, axis=0, keepdims=True)

    a_f = a_bf.astype(jnp.float32)
    deg = jnp.sum(a_f, axis=1, keepdims=True)
    inter = jax.lax.dot_general(a_bf, a_bf, (((1,), (1,)), ((), ())),
                                preferred_element_type=jnp.float32)
    sub = deg - inter - a_f * (1.0 - diag_row)

    ratio = (1.0 + _SIG1 * sub) / (1.0 + _SIG1 * inter)
    v = 1.0 / (1.0 + jnp.exp(ratio))

    u = m_cls * jnp.exp(-_GAMMA * preds)
    e = jnp.exp(_GAMMA * preds)
    t = jax.lax.dot_general(v, u, (((0,), (0,)), ((), ())),
                            preferred_element_type=jnp.float32)
    p = jax.lax.dot_general(t * e, m_cls, (((0,), (0,)), ((), ())),
                            preferred_element_type=jnp.float32)

    sub_pos = (sub > 0.0).astype(jnp.float32)
    inter_pos = (inter > 0.0).astype(jnp.float32)
    s_sub = jax.lax.dot_general(
        m_cls,
        jax.lax.dot_general(sub_pos, m_cls, (((1,), (0,)), ((), ())),
                            preferred_element_type=jnp.float32),
        (((0,), (0,)), ((), ())), preferred_element_type=jnp.float32)
    s_inter = jax.lax.dot_general(
        m_cls,
        jax.lax.dot_general(inter_pos, m_cls, (((1,), (0,)), ((), ())),
                            preferred_element_type=jnp.float32),
        (((0,), (0,)), ((), ())), preferred_element_type=jnp.float32)

    denom = jnp.reshape(ncnt, (_C, 1)) * ncnt
    recip = jnp.where(denom > 0.0, 1.0 / jnp.where(denom > 0.0, denom, 1.0), 0.0)
    ii = jax.lax.broadcasted_iota(jnp.int32, (_C, _C), 0)
    jj = jax.lax.broadcasted_iota(jnp.int32, (_C, _C), 1)
    keep = jnp.logical_and(jnp.logical_and(s_sub > 0.0, s_inter > 0.0), ii != jj)
    pair_loss = jnp.sum(jnp.where(keep, p * recip, 0.0))

    out_ref[...] = jnp.reshape(ce + _PER * pair_loss, (1, 1))


def kernel(preds, labels, mask, w_values_dict, adj_matrix):
    del w_values_dict
    adj_b = adj_matrix.astype(bool)
    a_bf = adj_b.astype(jnp.bfloat16)
    diag_row = jnp.diagonal(adj_b).astype(jnp.float32).reshape(1, _N)
    lab = labels.astype(jnp.int32).reshape(_N, 1)
    maskf = mask.astype(jnp.float32).reshape(_N, 1)
    out = pl.pallas_call(
        _loss_body,
        out_shape=jax.ShapeDtypeStruct((1, 1), jnp.float32),
    )(preds.astype(jnp.float32), lab, maskf, a_bf, diag_row)
    return out[0, 0]
